# baseline (speedup 1.0000x reference)
import functools

import jax
import jax.numpy as jnp
from jax import lax
from jax.experimental import pallas as pl
from jax.experimental.pallas import tpu as pltpu
from jax.experimental.pallas import tpu_sc as plsc

L = 8192
D_MODEL = 1024
LANES = 16
NC = 2
NS = 16
NW = NC * NS
TPW = L // NW
CHUNK = 32
NCHUNK = TPW // CHUNK


def _body(posx_hbm, posy_hbm, stab_hbm, idx_hbm, out_hbm,
          idxv, bufx, bufy, bufs, sem):
    wid = lax.axis_index("s") * NC + lax.axis_index("c")
    pltpu.sync_copy(idx_hbm.at[wid], idxv)

    def chunk_body(c, carry):
        cpx = pltpu.async_copy(posx_hbm.at[idxv.at[0, c]], bufx, sem)
        cpy = pltpu.async_copy(posy_hbm.at[idxv.at[1, c]], bufy, sem)
        cps = pltpu.async_copy(stab_hbm.at[idxv.at[2, c]], bufs, sem)
        cpx.wait()
        cpy.wait()
        cps.wait()

        def row_body(r, rcarry):
            for k in range(D_MODEL // LANES):
                sl = pl.ds(k * LANES, LANES)
                bufx[r, sl] = bufx[r, sl] + bufy[r, sl] + bufs[r, sl]
            return rcarry

        lax.fori_loop(0, CHUNK, row_body, 0, unroll=False)
        pltpu.sync_copy(bufx, out_hbm.at[pl.ds(wid * TPW + c * CHUNK, CHUNK)])
        return carry

    lax.fori_loop(0, NCHUNK, chunk_body, 0, unroll=False)


@jax.jit
def _pe_sum(pos_x, pos_y, stab, idx):
    mesh = plsc.VectorSubcoreMesh(core_axis_name="c", subcore_axis_name="s")
    f = pl.kernel(
        _body,
        out_type=jax.ShapeDtypeStruct((L, D_MODEL), jnp.float32),
        mesh=mesh,
        scratch_types=[
            pltpu.VMEM((3, NCHUNK, CHUNK), jnp.int32),
            pltpu.VMEM((CHUNK, D_MODEL), jnp.float32),
            pltpu.VMEM((CHUNK, D_MODEL), jnp.float32),
            pltpu.VMEM((CHUNK, D_MODEL), jnp.float32),
            pltpu.SemaphoreType.DMA,
        ],
    )
    return f(pos_x, pos_y, stab, idx)


def kernel(x, pos_x, pos_y, stab, token_to_x, token_to_y, token_to_stab):
    del x
    idx = jnp.stack([
        token_to_x[:L].astype(jnp.int32).reshape(NW, NCHUNK, CHUNK),
        token_to_y[:L].astype(jnp.int32).reshape(NW, NCHUNK, CHUNK),
        token_to_stab[:L].astype(jnp.int32).reshape(NW, NCHUNK, CHUNK),
    ], axis=1)
    return _pe_sum(pos_x, pos_y, stab, idx)


# speedup vs baseline: 2.8231x; 2.8231x over previous
import jax
import jax.numpy as jnp
from jax import lax
from jax.experimental import pallas as pl
from jax.experimental.pallas import tpu as pltpu
from jax.experimental.pallas import tpu_sc as plsc

L = 8192
D_MODEL = 1024
LANES = 16
NC = 2
NS = 16
NW = NC * NS
TPW = L // NW
CHUNK = 16
NCHUNK = TPW // CHUNK
NBUF = 3


def _body(posx_hbm, posy_hbm, idx_hbm, out_hbm,
          idxv, bufx0, bufy0, bufx1, bufy1, bufx2, bufy2, gsem, osem):
    wid = lax.axis_index("s") * NC + lax.axis_index("c")
    bufx = (bufx0, bufx1, bufx2)
    bufy = (bufy0, bufy1, bufy2)
    pltpu.sync_copy(idx_hbm.at[wid], idxv)

    def start_gather(c):
        s = c % NBUF
        gx = pltpu.async_copy(posx_hbm.at[idxv.at[0, c]], bufx[s], gsem)
        gy = pltpu.async_copy(posy_hbm.at[idxv.at[1, c]], bufy[s], gsem)
        return gx, gy

    outcps = [None] * NCHUNK
    gcps = [None] * NCHUNK
    gcps[0] = start_gather(0)
    for c in range(NCHUNK):
        s = c % NBUF
        if c >= 2:
            outcps[c - 2].wait()
        if c + 1 < NCHUNK:
            gcps[c + 1] = start_gather(c + 1)
        gx, gy = gcps[c]
        gx.wait()
        gy.wait()

        bx, by = bufx[s], bufy[s]

        def row_body(r, rcarry, bx=bx, by=by):
            for k in range(D_MODEL // LANES):
                sl = pl.ds(k * LANES, LANES)
                plsc.addupdate(bx.at[r, sl], by[r, sl])
            return rcarry

        lax.fori_loop(0, CHUNK, row_body, 0, unroll=False)
        cp = pltpu.make_async_copy(
            bx, out_hbm.at[pl.ds(wid * TPW + c * CHUNK, CHUNK)], osem)
        cp.start()
        outcps[c] = cp
    outcps[NCHUNK - 2].wait()
    outcps[NCHUNK - 1].wait()


@jax.jit
def _pe_sum(pos_x, pos_y_ext, idx):
    mesh = plsc.VectorSubcoreMesh(core_axis_name="c", subcore_axis_name="s")
    f = pl.kernel(
        _body,
        out_type=jax.ShapeDtypeStruct((L, D_MODEL), jnp.float32),
        mesh=mesh,
        scratch_types=[
            pltpu.VMEM((2, NCHUNK, CHUNK), jnp.int32),
            pltpu.VMEM((CHUNK, D_MODEL), jnp.float32),
            pltpu.VMEM((CHUNK, D_MODEL), jnp.float32),
            pltpu.VMEM((CHUNK, D_MODEL), jnp.float32),
            pltpu.VMEM((CHUNK, D_MODEL), jnp.float32),
            pltpu.VMEM((CHUNK, D_MODEL), jnp.float32),
            pltpu.VMEM((CHUNK, D_MODEL), jnp.float32),
            pltpu.SemaphoreType.DMA,
            pltpu.SemaphoreType.DMA,
        ],
    )
    return f(pos_x, pos_y_ext, idx)


def kernel(x, pos_x, pos_y, stab, token_to_x, token_to_y, token_to_stab):
    del x
    nrow = pos_y.shape[0]
    pos_y_ext = jnp.concatenate([pos_y + stab[0], pos_y + stab[1]], axis=0)
    tx = token_to_x[:L].astype(jnp.int32)
    tyx = (token_to_y[:L] + nrow * token_to_stab[:L]).astype(jnp.int32)
    idx = jnp.stack([
        tx.reshape(NW, NCHUNK, CHUNK),
        tyx.reshape(NW, NCHUNK, CHUNK),
    ], axis=1)
    return _pe_sum(pos_x, pos_y_ext, idx)


# speedup vs baseline: 2.9272x; 1.0369x over previous
import jax
import jax.numpy as jnp
from jax import lax
from jax.experimental import pallas as pl
from jax.experimental.pallas import tpu as pltpu
from jax.experimental.pallas import tpu_sc as plsc

L = 8192
D_MODEL = 1024
LANES = 16
NC = 2
NS = 16
NW = NC * NS
TPW = L // NW
CHUNK = 16
NCHUNK = TPW // CHUNK
NBX = 4
NBY = 3
PF = 2


def _body(posx_hbm, posy_hbm, idx_hbm, out_hbm,
          idxv, bufx0, bufx1, bufx2, bufx3, bufy0, bufy1, bufy2,
          gsem, osem):
    wid = lax.axis_index("s") * NC + lax.axis_index("c")
    bufx = (bufx0, bufx1, bufx2, bufx3)
    bufy = (bufy0, bufy1, bufy2)
    pltpu.sync_copy(idx_hbm.at[wid], idxv)

    def start_gather(c):
        gx = pltpu.async_copy(posx_hbm.at[idxv.at[0, c]], bufx[c % NBX], gsem)
        gy = pltpu.async_copy(posy_hbm.at[idxv.at[1, c]], bufy[c % NBY], gsem)
        return gx, gy

    outcps = [None] * NCHUNK
    gcps = [None] * NCHUNK
    for c in range(PF):
        gcps[c] = start_gather(c)
    for c in range(NCHUNK):
        if c + PF - NBX >= 0:
            outcps[c + PF - NBX].wait()
        if c + PF < NCHUNK:
            gcps[c + PF] = start_gather(c + PF)
        gx, gy = gcps[c]
        gx.wait()
        gy.wait()

        bx, by = bufx[c % NBX], bufy[c % NBY]

        def row_body(r, rcarry, bx=bx, by=by):
            for k in range(D_MODEL // LANES):
                sl = pl.ds(k * LANES, LANES)
                plsc.addupdate(bx.at[r, sl], by[r, sl])
            return rcarry

        lax.fori_loop(0, CHUNK, row_body, 0, unroll=False)
        cp = pltpu.make_async_copy(
            bx, out_hbm.at[pl.ds(wid * TPW + c * CHUNK, CHUNK)], osem)
        cp.start()
        outcps[c] = cp
    for c in range(max(0, NCHUNK - (NBX - PF)), NCHUNK):
        outcps[c].wait()


@jax.jit
def _pe_sum(pos_x, pos_y_ext, idx):
    mesh = plsc.VectorSubcoreMesh(core_axis_name="c", subcore_axis_name="s")
    f = pl.kernel(
        _body,
        out_type=jax.ShapeDtypeStruct((L, D_MODEL), jnp.float32),
        mesh=mesh,
        scratch_types=[
            pltpu.VMEM((2, NCHUNK, CHUNK), jnp.int32),
            pltpu.VMEM((CHUNK, D_MODEL), jnp.float32),
            pltpu.VMEM((CHUNK, D_MODEL), jnp.float32),
            pltpu.VMEM((CHUNK, D_MODEL), jnp.float32),
            pltpu.VMEM((CHUNK, D_MODEL), jnp.float32),
            pltpu.VMEM((CHUNK, D_MODEL), jnp.float32),
            pltpu.VMEM((CHUNK, D_MODEL), jnp.float32),
            pltpu.VMEM((CHUNK, D_MODEL), jnp.float32),
            pltpu.SemaphoreType.DMA,
            pltpu.SemaphoreType.DMA,
        ],
    )
    return f(pos_x, pos_y_ext, idx)


def kernel(x, pos_x, pos_y, stab, token_to_x, token_to_y, token_to_stab):
    del x
    nrow = pos_y.shape[0]
    pos_y_ext = jnp.concatenate([pos_y + stab[0], pos_y + stab[1]], axis=0)
    tx = token_to_x[:L].astype(jnp.int32)
    tyx = (token_to_y[:L] + nrow * token_to_stab[:L]).astype(jnp.int32)
    idx = jnp.stack([
        tx.reshape(NW, NCHUNK, CHUNK),
        tyx.reshape(NW, NCHUNK, CHUNK),
    ], axis=1)
    return _pe_sum(pos_x, pos_y_ext, idx)


# speedup vs baseline: 3.2451x; 1.1086x over previous
import jax
import jax.numpy as jnp
from jax import lax
from jax.experimental import pallas as pl
from jax.experimental.pallas import tpu as pltpu
from jax.experimental.pallas import tpu_sc as plsc

L = 8192
D_MODEL = 1024
LANES = 16
NC = 2
NS = 16
NW = NC * NS
TPW = L // NW
CHUNK = 16
NCHUNK = TPW // CHUNK
NBX = 4
NBY = 3
PF = 2


def _body(posx_hbm, posy_hbm, idx_hbm, out_hbm,
          idxv, bufx0, bufx1, bufx2, bufx3, bufy0, bufy1, bufy2,
          gsem, osem):
    wid = lax.axis_index("s") * NC + lax.axis_index("c")
    bufx = (bufx0, bufx1, bufx2, bufx3)
    bufy = (bufy0, bufy1, bufy2)
    pltpu.sync_copy(idx_hbm.at[wid], idxv)

    def start_gather(c):
        gx = pltpu.async_copy(posx_hbm.at[idxv.at[0, c]], bufx[c % NBX], gsem)
        gy = pltpu.async_copy(posy_hbm.at[idxv.at[1, c]], bufy[c % NBY], gsem)
        return gx, gy

    outcps = [None] * NCHUNK
    gcps = [None] * NCHUNK
    for c in range(PF):
        gcps[c] = start_gather(c)
    for c in range(NCHUNK):
        if c + PF - NBX >= 0:
            outcps[c + PF - NBX].wait()
        if c + PF < NCHUNK:
            gcps[c + PF] = start_gather(c + PF)
        gx, gy = gcps[c]
        gx.wait()
        gy.wait()

        bx, by = bufx[c % NBX], bufy[c % NBY]

        def col_body(k, kcarry, bx=bx, by=by):
            sl = pl.ds(k * LANES, LANES)
            for r in range(CHUNK):
                plsc.addupdate(bx.at[r, sl], by[r, sl])
            return kcarry

        lax.fori_loop(0, D_MODEL // LANES, col_body, 0, unroll=False)
        cp = pltpu.make_async_copy(
            bx, out_hbm.at[pl.ds(wid * TPW + c * CHUNK, CHUNK)], osem)
        cp.start()
        outcps[c] = cp
    for c in range(max(0, NCHUNK - (NBX - PF)), NCHUNK):
        outcps[c].wait()


@jax.jit
def _pe_sum(pos_x, pos_y_ext, idx):
    mesh = plsc.VectorSubcoreMesh(core_axis_name="c", subcore_axis_name="s")
    f = pl.kernel(
        _body,
        out_type=jax.ShapeDtypeStruct((L, D_MODEL), jnp.float32),
        mesh=mesh,
        scratch_types=[
            pltpu.VMEM((2, NCHUNK, CHUNK), jnp.int32),
            pltpu.VMEM((CHUNK, D_MODEL), jnp.float32),
            pltpu.VMEM((CHUNK, D_MODEL), jnp.float32),
            pltpu.VMEM((CHUNK, D_MODEL), jnp.float32),
            pltpu.VMEM((CHUNK, D_MODEL), jnp.float32),
            pltpu.VMEM((CHUNK, D_MODEL), jnp.float32),
            pltpu.VMEM((CHUNK, D_MODEL), jnp.float32),
            pltpu.VMEM((CHUNK, D_MODEL), jnp.float32),
            pltpu.SemaphoreType.DMA,
            pltpu.SemaphoreType.DMA,
        ],
    )
    return f(pos_x, pos_y_ext, idx)


def kernel(x, pos_x, pos_y, stab, token_to_x, token_to_y, token_to_stab):
    del x
    nrow = pos_y.shape[0]
    pos_y_ext = jnp.concatenate([pos_y + stab[0], pos_y + stab[1]], axis=0)
    tx = token_to_x[:L].astype(jnp.int32)
    tyx = (token_to_y[:L] + nrow * token_to_stab[:L]).astype(jnp.int32)
    idx = jnp.stack([
        tx.reshape(NW, NCHUNK, CHUNK),
        tyx.reshape(NW, NCHUNK, CHUNK),
    ], axis=1)
    return _pe_sum(pos_x, pos_y_ext, idx)
